# Initial kernel scaffold; baseline (speedup 1.0000x reference)
import jax
import jax.numpy as jnp
from jax.experimental import pallas as pl


def kernel(input_ids, label_ids, item_table, label_table, pos_table):
    raise NotImplementedError("write your pallas kernel here")


# speedup vs baseline: 8.2023x; 8.2023x over previous
import functools

import jax
import jax.numpy as jnp
from jax import lax
from jax.experimental import pallas as pl
from jax.experimental.pallas import tpu as pltpu
from jax.experimental.pallas import tpu_sc as plsc

BATCH = 4096
SEQ = 200
EMBED = 128
LANES = 16
NVEC = EMBED // LANES

CHUNKS = ((0, 128), (128, 72))


def _make_kernel(num_cores, num_subcores):
    nw = num_cores * num_subcores
    b_per_w = BATCH // nw

    mesh = plsc.VectorSubcoreMesh(core_axis_name="c", subcore_axis_name="s")

    @functools.partial(
        pl.kernel,
        mesh=mesh,
        out_type=jax.ShapeDtypeStruct((BATCH, SEQ, EMBED), jnp.float32),
        scratch_types=[
            pltpu.VMEM((b_per_w, SEQ), jnp.int32),
            pltpu.VMEM((b_per_w,), jnp.int32),
            pltpu.VMEM((b_per_w, EMBED), jnp.float32),
            pltpu.VMEM((SEQ, EMBED), jnp.float32),
            pltpu.VMEM((128, EMBED), jnp.float32),
            pltpu.SemaphoreType.DMA,
        ],
    )
    def k(ids_hbm, labels_hbm, item_hbm, ltab_hbm, pos_hbm, out_hbm,
          ids_v, labs_v, user_v, pos_v, buf, sem):
        wid = lax.axis_index("s") * num_cores + lax.axis_index("c")
        b0 = wid * b_per_w

        pltpu.sync_copy(ids_hbm.at[pl.ds(b0, b_per_w)], ids_v)
        pltpu.sync_copy(labels_hbm.at[pl.ds(b0, b_per_w)], labs_v)
        pltpu.sync_copy(pos_hbm, pos_v)
        pltpu.async_copy(ltab_hbm.at[labs_v], user_v, sem).wait()

        def batch_body(bl, carry):
            user_vecs = [user_v[bl, pl.ds(LANES * j, LANES)] for j in range(NVEC)]
            for off, n in CHUNKS:
                idx = ids_v.at[bl, pl.ds(off, n)]
                dst = buf.at[pl.ds(0, n)]
                pltpu.async_copy(item_hbm.at[idx], dst, sem).wait()

                def row_body(i, c, off=off):
                    for j in range(NVEC):
                        sl = pl.ds(LANES * j, LANES)
                        buf[i, sl] = buf[i, sl] + pos_v[off + i, sl] + user_vecs[j]
                    return c

                lax.fori_loop(0, n, row_body, 0)
                pltpu.sync_copy(dst, out_hbm.at[b0 + bl, pl.ds(off, n)])
            return carry

        lax.fori_loop(0, b_per_w, batch_body, 0)

    return k


def kernel(input_ids, label_ids, item_table, label_table, pos_table):
    info = plsc.get_sparse_core_info()
    k = _make_kernel(info.num_cores, info.num_subcores)
    return k(input_ids.astype(jnp.int32), label_ids.astype(jnp.int32),
             item_table, label_table, pos_table)
